# baseline (speedup 1.0000x reference)
import jax
import jax.numpy as jnp
from jax import lax
from jax.experimental import pallas as pl
from jax.experimental.pallas import tpu as pltpu
from jax.experimental.pallas import tpu_sc as plsc

B = 64
T = 128
H = 256
W = 256
C = 4
NC = 2
NS = 16
L = 16
NW = NC * NS
NCHUNK = T // L


def _bin_chunk(xy_v, k, b):
    x = xy_v[0, pl.ds(k * L, L)]
    y = xy_v[1, pl.ds(k * L, L)]
    gx = (((x + 10.0) / 20.0) * float(H)).astype(jnp.int32)
    gy = (((y + 10.0) / 20.0) * float(W)).astype(jnp.int32)
    inb = (gx >= 0) & (gx < H) & (gy >= 0) & (gy < W)
    ic = jnp.clip(gx, 0, H - 1)
    jc = jnp.clip(gy, 0, W - 1)
    e0 = (b * 512 + ic * 2 + (jc >> 7)) * 512 + (jc & 127)
    return inb, e0


def _batch_indices(xy_v, idx_v, b):
    def body(k, carry):
        _, e0 = _bin_chunk(xy_v, k, b)
        for c in range(C):
            idx_v[c, pl.ds(k * L, L)] = e0 + c * 128
        return carry
    lax.fori_loop(0, NCHUNK, body, 0, unroll=False)


def _reduce_batch(dst_v, xy_v, b):
    def body(k, acc):
        inb, _ = _bin_chunk(xy_v, k, b)
        mass = dst_v[0, pl.ds(k * L, L)]
        for c in range(1, C):
            mass = mass + dst_v[c, pl.ds(k * L, L)]
        ok = inb & jnp.logical_not(mass > 100.0)
        return acc & ok.astype(jnp.int32)
    acc = lax.fori_loop(0, NCHUNK, body, jnp.ones((L,), jnp.int32),
                        unroll=False)
    return jnp.min(acc)


def _collision_body(traj_hbm, map_hbm, out_hbm,
                    xy_a, xy_b, idx_a, idx_b, dst_a, dst_b,
                    res_v, stage_v, res8_v, shared, sem_a, sem_b, sem_t):
    cid = lax.axis_index("c")
    sid = lax.axis_index("s")
    b0 = cid * 2 * NS + sid
    b1 = b0 + NS

    ta = pltpu.async_copy(traj_hbm.at[pl.ds(0, 2), b0], xy_a, sem_t)
    tb = pltpu.async_copy(traj_hbm.at[pl.ds(0, 2), b1], xy_b, sem_t)
    ta.wait()
    _batch_indices(xy_a, idx_a, b0)
    ga = [pltpu.async_copy(map_hbm.at[idx_a.at[c]], dst_a.at[c], sem_a)
          for c in range(C)]

    tb.wait()
    _batch_indices(xy_b, idx_b, b1)
    gb = [pltpu.async_copy(map_hbm.at[idx_b.at[c]], dst_b.at[c], sem_b)
          for c in range(C)]

    for g in ga:
        g.wait()
    v0 = _reduce_batch(dst_a, xy_a, b0)
    for g in gb:
        g.wait()
    v1 = _reduce_batch(dst_b, xy_b, b1)

    lane = lax.iota(jnp.int32, L)
    row = jnp.where(lane == 0, jnp.broadcast_to(v0, (L,)),
                    jnp.where(lane == 1, jnp.broadcast_to(v1, (L,)), 0))
    res_v[...] = row
    pltpu.sync_copy(res_v, shared.at[sid])
    plsc.subcore_barrier()

    @pl.when(sid == 0)
    def _pack():
        pltpu.sync_copy(shared, stage_v)
        g0 = plsc.load_gather(stage_v, [lane, jnp.zeros((L,), jnp.int32)])
        g1 = plsc.load_gather(stage_v, [lane, jnp.full((L,), 1, jnp.int32)])
        res8_v[pl.ds(0, L)] = g0
        res8_v[pl.ds(L, L)] = g1
        pltpu.sync_copy(res8_v, out_hbm.at[pl.ds(cid * 32, 32)])


@jax.jit
def _collision_sc(traj_planes, map_flat):
    kfn = pl.kernel(
        _collision_body,
        out_type=jax.ShapeDtypeStruct((B,), jnp.int32),
        mesh=plsc.VectorSubcoreMesh(
            core_axis_name="c", subcore_axis_name="s",
            num_cores=NC, num_subcores=NS),
        scratch_types=[
            pltpu.VMEM((2, T), jnp.float32),
            pltpu.VMEM((2, T), jnp.float32),
            pltpu.VMEM((C, T), jnp.int32),
            pltpu.VMEM((C, T), jnp.int32),
            pltpu.VMEM((C, T), jnp.float32),
            pltpu.VMEM((C, T), jnp.float32),
            pltpu.VMEM((L,), jnp.int32),
            pltpu.VMEM((NS, L), jnp.int32),
            pltpu.VMEM((2 * L,), jnp.int32),
            pltpu.VMEM_SHARED((NS, L), jnp.int32),
            pltpu.SemaphoreType.DMA,
            pltpu.SemaphoreType.DMA,
            pltpu.SemaphoreType.DMA,
        ],
        compiler_params=pltpu.CompilerParams(needs_layout_passes=False),
    )
    return kfn(traj_planes, map_flat)


def kernel(trajectory, affordance_map):
    traj_planes = trajectory.transpose(2, 0, 1)
    map_flat = (affordance_map
                .reshape(B, H, 2, W // 2, C)
                .transpose(0, 1, 2, 4, 3)
                .reshape(B * H * W * C))
    out = _collision_sc(traj_planes, map_flat)
    return out.astype(jnp.bool_)


# speedup vs baseline: 1.0032x; 1.0032x over previous
import jax
import jax.numpy as jnp
from jax import lax
from jax.experimental import pallas as pl
from jax.experimental.pallas import tpu as pltpu
from jax.experimental.pallas import tpu_sc as plsc

B = 64
T = 128
H = 256
W = 256
C = 4
NC = 2
NS = 16
L = 16
NW = NC * NS
NCHUNK = T // L


def _batch_indices(xy_v, idx_v, b):
    in_bounds = []
    for k in range(NCHUNK):
        x = xy_v[0, pl.ds(k * L, L)]
        y = xy_v[1, pl.ds(k * L, L)]
        gx = (((x + 10.0) / 20.0) * float(H)).astype(jnp.int32)
        gy = (((y + 10.0) / 20.0) * float(W)).astype(jnp.int32)
        inb = (gx >= 0) & (gx < H) & (gy >= 0) & (gy < W)
        ic = jnp.clip(gx, 0, H - 1)
        jc = jnp.clip(gy, 0, W - 1)
        e0 = (b * 512 + ic * 2 + (jc >> 7)) * 512 + (jc & 127)
        for c in range(C):
            idx_v[c, pl.ds(k * L, L)] = e0 + c * 128
        in_bounds.append(inb)
    return in_bounds


def _reduce_batch(dst_v, in_bounds):
    acc = None
    for k in range(NCHUNK):
        mass = dst_v[0, pl.ds(k * L, L)]
        for c in range(1, C):
            mass = mass + dst_v[c, pl.ds(k * L, L)]
        ok = in_bounds[k] & jnp.logical_not(mass > 100.0)
        acc = ok if acc is None else (acc & ok)
    return jnp.all(acc).astype(jnp.int32)


def _collision_body(traj_hbm, map_hbm, out_hbm,
                    xy_v, idx_v, dst_v,
                    res_v, stage_v, res8_v, shared, sem_a, sem_b):
    cid = lax.axis_index("c")
    sid = lax.axis_index("s")
    b0 = cid * 2 * NS + sid
    b1 = b0 + NS
    xy_a, xy_b = xy_v.at[0], xy_v.at[1]
    idx_a, idx_b = idx_v.at[0], idx_v.at[1]
    dst_a, dst_b = dst_v.at[0], dst_v.at[1]

    ta = pltpu.async_copy(traj_hbm.at[pl.ds(0, 2), b0], xy_a, sem_a)
    tb = pltpu.async_copy(traj_hbm.at[pl.ds(0, 2), b1], xy_b, sem_b)
    ta.wait()
    inb_a = _batch_indices(xy_a, idx_a, b0)
    ga = [pltpu.async_copy(map_hbm.at[idx_a.at[c]], dst_a.at[c], sem_a)
          for c in range(C)]

    tb.wait()
    inb_b = _batch_indices(xy_b, idx_b, b1)
    gb = [pltpu.async_copy(map_hbm.at[idx_b.at[c]], dst_b.at[c], sem_b)
          for c in range(C)]

    for g in ga:
        g.wait()
    v0 = _reduce_batch(dst_a, inb_a)
    for g in gb:
        g.wait()
    v1 = _reduce_batch(dst_b, inb_b)

    lane = lax.iota(jnp.int32, L)
    row = jnp.where(lane == 0, jnp.broadcast_to(v0, (L,)),
                    jnp.where(lane == 1, jnp.broadcast_to(v1, (L,)), 0))
    res_v[...] = row
    pltpu.sync_copy(res_v, shared.at[sid])
    plsc.subcore_barrier()

    @pl.when(sid == 0)
    def _pack():
        pltpu.sync_copy(shared, stage_v)
        g0 = plsc.load_gather(stage_v, [lane, jnp.zeros((L,), jnp.int32)])
        g1 = plsc.load_gather(stage_v, [lane, jnp.full((L,), 1, jnp.int32)])
        res8_v[pl.ds(0, L)] = g0
        res8_v[pl.ds(L, L)] = g1
        pltpu.sync_copy(res8_v, out_hbm.at[pl.ds(cid * 32, 32)])


@jax.jit
def _collision_sc(traj_planes, map_flat):
    kfn = pl.kernel(
        _collision_body,
        out_type=jax.ShapeDtypeStruct((B,), jnp.int32),
        mesh=plsc.VectorSubcoreMesh(
            core_axis_name="c", subcore_axis_name="s",
            num_cores=NC, num_subcores=NS),
        scratch_types=[
            pltpu.VMEM((2, 2, T), jnp.float32),
            pltpu.VMEM((2, C, T), jnp.int32),
            pltpu.VMEM((2, C, T), jnp.float32),
            pltpu.VMEM((L,), jnp.int32),
            pltpu.VMEM((NS, L), jnp.int32),
            pltpu.VMEM((2 * L,), jnp.int32),
            pltpu.VMEM_SHARED((NS, L), jnp.int32),
            pltpu.SemaphoreType.DMA,
            pltpu.SemaphoreType.DMA,
        ],
        compiler_params=pltpu.CompilerParams(needs_layout_passes=False),
    )
    return kfn(traj_planes, map_flat)


def kernel(trajectory, affordance_map):
    traj_planes = trajectory.transpose(2, 0, 1)
    map_flat = (affordance_map
                .reshape(B, H, 2, W // 2, C)
                .transpose(0, 1, 2, 4, 3)
                .reshape(B * H * W * C))
    out = _collision_sc(traj_planes, map_flat)
    return out.astype(jnp.bool_)
